# baseline (speedup 1.0000x reference)
import functools

import jax
import jax.numpy as jnp
from jax.experimental import pallas as pl
from jax.experimental.pallas import tpu as pltpu

_NP = 20000
_NG = 1000
_C = 256
_BP = 1000
_NB = _NP // _BP


def _iou_inputs(pbox_ref, gt_ref):
    px1 = pbox_ref[:, 0:1]
    py1 = pbox_ref[:, 1:2]
    px2 = pbox_ref[:, 2:3]
    py2 = pbox_ref[:, 3:4]
    gx1 = gt_ref[0:1, :]
    gy1 = gt_ref[1:2, :]
    gx2 = gt_ref[2:3, :]
    gy2 = gt_ref[3:4, :]
    wx = jnp.maximum(jnp.minimum(px2, gx2) - jnp.maximum(px1, gx1), 0.0)
    wy = jnp.maximum(jnp.minimum(py2, gy2) - jnp.maximum(py1, gy1), 0.0)
    inter = wx * wy
    areas = (px2 - px1) * (py2 - py1) + (gx2 - gx1) * (gy2 - gy1)
    coords = ((px1, gx1), (py1, gy1), (px2, gx2), (py2, gy2))
    return inter, areas, coords


def _lse(logits_ref):
    x = logits_ref[...]
    rowmax = jnp.max(x, axis=1, keepdims=True)
    return x, rowmax + jnp.log(
        jnp.sum(jnp.exp(x - rowmax), axis=1, keepdims=True))


def _pick_matmul(x, labels_ref):
    lab = labels_ref[0:1, :]
    onehot = (jax.lax.broadcasted_iota(jnp.int32, (_C, _NG), 0) == lab
              ).astype(jnp.bfloat16)
    return jax.lax.dot_general(
        x.astype(jnp.bfloat16), onehot,
        dimension_numbers=(((1,), (0,)), ((), ())),
        preferred_element_type=jnp.float32)


def _sl1_raw(coords):
    s_raw = None
    for pk, gk in coords:
        ad = jnp.abs(pk - gk)
        m = jnp.minimum(ad, ad.dtype.type(1))
        t = m * (ad + ad - m)
        s_raw = t if s_raw is None else s_raw + t
    return s_raw


def _main_body(labels_ref, gt_ref, pbox_ref, logits_ref, labc_ref,
               out_ref, cnt_out_ref,
               cnt_ref, pick_ref, lsem_ref, sl1_ref, onehot_ref):
    i = pl.program_id(0)

    @pl.when(i == 0)
    def _init():
        cnt_ref[...] = jnp.zeros_like(cnt_ref)
        pick_ref[...] = jnp.zeros_like(pick_ref)
        lsem_ref[...] = jnp.zeros_like(lsem_ref)
        sl1_ref[...] = jnp.zeros_like(sl1_ref)
        labc = labc_ref[:, 0:1]
        onehot_ref[...] = (
            jax.lax.broadcasted_iota(jnp.int32, (_NG, _C), 1) == labc
        ).astype(jnp.bfloat16)

    x, lse = _lse(logits_ref)

    bf = jnp.bfloat16
    px1 = pbox_ref[:, 0:1].astype(bf)
    py1 = pbox_ref[:, 1:2].astype(bf)
    px2 = pbox_ref[:, 2:3].astype(bf)
    py2 = pbox_ref[:, 3:4].astype(bf)
    gx1 = gt_ref[0:1, :].astype(bf)
    gy1 = gt_ref[1:2, :].astype(bf)
    gx2 = gt_ref[2:3, :].astype(bf)
    gy2 = gt_ref[3:4, :].astype(bf)
    wx = jnp.maximum(jnp.minimum(px2, gx2) - jnp.maximum(px1, gx1), bf(0))
    wy = jnp.maximum(jnp.minimum(py2, gy2) - jnp.maximum(py1, gy1), bf(0))
    inter = wx * wy
    areas = (px2 - px1) * (py2 - py1) + (gx2 - gx1) * (gy2 - gy1)
    cmp = inter * bf(13.0 / 3.0) > areas
    mask_bf = cmp.astype(bf)

    cnt_mat = jax.lax.dot_general(
        mask_bf, onehot_ref[...],
        dimension_numbers=(((1,), (0,)), ((), ())),
        preferred_element_type=jnp.float32)

    rowcnt = jnp.sum(cnt_mat, axis=1, keepdims=True)
    cnt_ref[...] += jnp.sum(rowcnt, keepdims=True)
    pick_ref[...] += jnp.sum(cnt_mat * x, keepdims=True)
    lsem_ref[...] += jnp.sum(rowcnt * lse, keepdims=True)

    s_raw = _sl1_raw(((px1, gx1), (py1, gy1), (px2, gx2), (py2, gy2)))
    s_masked = jnp.where(cmp, s_raw, bf(0))
    srow = jax.lax.dot_general(
        s_masked, jnp.ones((_NG, 128), bf),
        dimension_numbers=(((1,), (0,)), ((), ())),
        preferred_element_type=jnp.float32)
    sl1_ref[...] += 0.5 * jnp.sum(srow[:, 0:1], keepdims=True)

    @pl.when(i == _NB - 1)
    def _finalize():
        count = cnt_ref[...]
        out_ref[...] = ((lsem_ref[...] - pick_ref[...]) / count
                        + sl1_ref[...] / (4.0 * count))
        cnt_out_ref[...] = count


def _fb_body(labels_ref, gt_ref, pbox_ref, logits_ref, labc_ref, out_ref,
             fbmax_ref, fbce_ref):
    i = pl.program_id(0)

    @pl.when(i == 0)
    def _init():
        fbmax_ref[...] = jnp.full_like(fbmax_ref, -1.0)
        fbce_ref[...] = jnp.zeros_like(fbce_ref)

    x, lse = _lse(logits_ref)
    inter, areas, coords = _iou_inputs(pbox_ref, gt_ref)
    iou = inter / (areas - inter)
    p_mat = _pick_matmul(x, labels_ref)
    s_raw = _sl1_raw(coords)

    bmax = jnp.max(iou, axis=0, keepdims=True)
    ridx = jax.lax.broadcasted_iota(jnp.int32, (_BP, _NG), 0)
    cand_rows = jnp.where(iou == bmax, ridx, _BP)
    minidx = jnp.min(cand_rows, axis=0, keepdims=True)
    sel = (ridx == minidx).astype(jnp.float32)
    cand = jnp.sum(sel * ((lse - p_mat) + 0.125 * s_raw),
                   axis=0, keepdims=True)
    prev = fbmax_ref[...]
    upd = bmax > prev
    fbce_ref[...] = jnp.where(upd, cand, fbce_ref[...])
    fbmax_ref[...] = jnp.where(upd, bmax, prev)

    @pl.when(i == _NB - 1)
    def _finalize():
        keep = (fbmax_ref[...] > 0.0).astype(jnp.float32)
        dfb = jnp.sum(keep, keepdims=True)
        out_ref[...] = jnp.sum(keep * fbce_ref[...], keepdims=True) / dfb


_IN_SPECS = [
    pl.BlockSpec((8, _NG), lambda i: (0, 0)),
    pl.BlockSpec((8, _NG), lambda i: (0, 0)),
    pl.BlockSpec((_BP, 4), lambda i: (i, 0)),
    pl.BlockSpec((_BP, _C), lambda i: (i, 0)),
    pl.BlockSpec((_NG, 8), lambda i: (0, 0)),
]


@functools.partial(jax.jit, static_argnames=())
def kernel(pred_class_logits, pred_bounding_boxes, gt_class, gt_bounding_boxes):
    labels = jnp.broadcast_to(
        gt_class[0].astype(jnp.int32)[None, :], (8, _NG))
    gt_t = jnp.zeros((8, _NG), jnp.float32).at[:4].set(gt_bounding_boxes[0].T)
    labc = jnp.broadcast_to(
        gt_class[0].astype(jnp.int32)[:, None], (_NG, 8))
    args = (labels, gt_t, pred_bounding_boxes, pred_class_logits, labc)

    main, count = pl.pallas_call(
        _main_body,
        grid=(_NB,),
        in_specs=_IN_SPECS,
        out_specs=[pl.BlockSpec((1, 1), lambda i: (0, 0)),
                   pl.BlockSpec((1, 1), lambda i: (0, 0))],
        out_shape=[jax.ShapeDtypeStruct((1, 1), jnp.float32),
                   jax.ShapeDtypeStruct((1, 1), jnp.float32)],
        scratch_shapes=[pltpu.VMEM((1, 1), jnp.float32)] * 4
        + [pltpu.VMEM((_NG, _C), jnp.bfloat16)],
    )(*args)

    def _fallback(_):
        fb = pl.pallas_call(
            _fb_body,
            grid=(_NB,),
            in_specs=_IN_SPECS,
            out_specs=pl.BlockSpec((1, 1), lambda i: (0, 0)),
            out_shape=jax.ShapeDtypeStruct((1, 1), jnp.float32),
            scratch_shapes=[pltpu.VMEM((1, _NG), jnp.float32)] * 2,
        )(*args)
        return fb[0, 0]

    return jax.lax.cond(count[0, 0] > 0.0, lambda _: main[0, 0],
                        _fallback, None)


# speedup vs baseline: 1.0077x; 1.0077x over previous
import functools

import jax
import jax.numpy as jnp
from jax.experimental import pallas as pl
from jax.experimental.pallas import tpu as pltpu

_NP = 20000
_NG = 1000
_C = 256
_BP = 1000
_NB = _NP // _BP


def _iou_inputs(pbox_ref, gt_ref):
    px1 = pbox_ref[:, 0:1]
    py1 = pbox_ref[:, 1:2]
    px2 = pbox_ref[:, 2:3]
    py2 = pbox_ref[:, 3:4]
    gx1 = gt_ref[0:1, :]
    gy1 = gt_ref[1:2, :]
    gx2 = gt_ref[2:3, :]
    gy2 = gt_ref[3:4, :]
    wx = jnp.maximum(jnp.minimum(px2, gx2) - jnp.maximum(px1, gx1), 0.0)
    wy = jnp.maximum(jnp.minimum(py2, gy2) - jnp.maximum(py1, gy1), 0.0)
    inter = wx * wy
    areas = (px2 - px1) * (py2 - py1) + (gx2 - gx1) * (gy2 - gy1)
    coords = ((px1, gx1), (py1, gy1), (px2, gx2), (py2, gy2))
    return inter, areas, coords


def _lse(logits_ref):
    x = logits_ref[...]
    rowmax = jnp.max(x, axis=1, keepdims=True)
    return x, rowmax + jnp.log(
        jnp.sum(jnp.exp(x - rowmax), axis=1, keepdims=True))


def _pick_matmul(x, labc_ref):
    labc = labc_ref[:, 0:1]
    onehot_g = (jax.lax.broadcasted_iota(jnp.int32, (_NG, _C), 1) == labc
                ).astype(jnp.bfloat16)
    return jax.lax.dot_general(
        x.astype(jnp.bfloat16), onehot_g,
        dimension_numbers=(((1,), (1,)), ((), ())),
        preferred_element_type=jnp.float32)


def _sl1_raw(coords):
    s_raw = None
    for pk, gk in coords:
        ad = jnp.abs(pk - gk)
        m = jnp.minimum(ad, ad.dtype.type(1))
        t = m * (ad + ad - m)
        s_raw = t if s_raw is None else s_raw + t
    return s_raw


def _main_body(gt_ref, pbox_ref, logits_ref, labc_ref,
               out_ref, cnt_out_ref,
               cnt_ref, pick_ref, lsem_ref, sl1_ref, onehot_ref):
    i = pl.program_id(0)

    @pl.when(i == 0)
    def _init():
        cnt_ref[...] = jnp.zeros_like(cnt_ref)
        pick_ref[...] = jnp.zeros_like(pick_ref)
        lsem_ref[...] = jnp.zeros_like(lsem_ref)
        sl1_ref[...] = jnp.zeros_like(sl1_ref)
        labc = labc_ref[:, 0:1]
        onehot_ref[...] = (
            jax.lax.broadcasted_iota(jnp.int32, (_NG, _C), 1) == labc
        ).astype(jnp.bfloat16)

    x, lse = _lse(logits_ref)

    bf = jnp.bfloat16
    px1 = pbox_ref[:, 0:1].astype(bf)
    py1 = pbox_ref[:, 1:2].astype(bf)
    px2 = pbox_ref[:, 2:3].astype(bf)
    py2 = pbox_ref[:, 3:4].astype(bf)
    gx1 = gt_ref[0:1, :].astype(bf)
    gy1 = gt_ref[1:2, :].astype(bf)
    gx2 = gt_ref[2:3, :].astype(bf)
    gy2 = gt_ref[3:4, :].astype(bf)
    wx = jnp.maximum(jnp.minimum(px2, gx2) - jnp.maximum(px1, gx1), bf(0))
    wy = jnp.maximum(jnp.minimum(py2, gy2) - jnp.maximum(py1, gy1), bf(0))
    inter = wx * wy
    areas = (px2 - px1) * (py2 - py1) + (gx2 - gx1) * (gy2 - gy1)
    cmp = inter * bf(13.0 / 3.0) > areas
    mask_bf = cmp.astype(bf)

    cnt_mat = jax.lax.dot_general(
        mask_bf, onehot_ref[...],
        dimension_numbers=(((1,), (0,)), ((), ())),
        preferred_element_type=jnp.float32)

    rowcnt = jnp.sum(cnt_mat, axis=1, keepdims=True)
    cnt_ref[...] += jnp.sum(rowcnt, keepdims=True)
    pick_ref[...] += jnp.sum(cnt_mat * x, keepdims=True)
    lsem_ref[...] += jnp.sum(rowcnt * lse, keepdims=True)

    s_raw = _sl1_raw(((px1, gx1), (py1, gy1), (px2, gx2), (py2, gy2)))
    s_masked = jnp.where(cmp, s_raw, bf(0))
    srow = jax.lax.dot_general(
        s_masked, jnp.ones((_NG, 128), bf),
        dimension_numbers=(((1,), (0,)), ((), ())),
        preferred_element_type=jnp.float32)
    sl1_ref[...] += 0.5 * jnp.sum(srow[:, 0:1], keepdims=True)

    @pl.when(i == _NB - 1)
    def _finalize():
        count = cnt_ref[...]
        out_ref[...] = ((lsem_ref[...] - pick_ref[...]) / count
                        + sl1_ref[...] / (4.0 * count))
        cnt_out_ref[...] = count


def _fb_body(gt_ref, pbox_ref, logits_ref, labc_ref, out_ref,
             fbmax_ref, fbce_ref):
    i = pl.program_id(0)

    @pl.when(i == 0)
    def _init():
        fbmax_ref[...] = jnp.full_like(fbmax_ref, -1.0)
        fbce_ref[...] = jnp.zeros_like(fbce_ref)

    x, lse = _lse(logits_ref)
    inter, areas, coords = _iou_inputs(pbox_ref, gt_ref)
    iou = inter / (areas - inter)
    p_mat = _pick_matmul(x, labc_ref)
    s_raw = _sl1_raw(coords)

    bmax = jnp.max(iou, axis=0, keepdims=True)
    ridx = jax.lax.broadcasted_iota(jnp.int32, (_BP, _NG), 0)
    cand_rows = jnp.where(iou == bmax, ridx, _BP)
    minidx = jnp.min(cand_rows, axis=0, keepdims=True)
    sel = (ridx == minidx).astype(jnp.float32)
    cand = jnp.sum(sel * ((lse - p_mat) + 0.125 * s_raw),
                   axis=0, keepdims=True)
    prev = fbmax_ref[...]
    upd = bmax > prev
    fbce_ref[...] = jnp.where(upd, cand, fbce_ref[...])
    fbmax_ref[...] = jnp.where(upd, bmax, prev)

    @pl.when(i == _NB - 1)
    def _finalize():
        keep = (fbmax_ref[...] > 0.0).astype(jnp.float32)
        dfb = jnp.sum(keep, keepdims=True)
        out_ref[...] = jnp.sum(keep * fbce_ref[...], keepdims=True) / dfb


_IN_SPECS = [
    pl.BlockSpec((8, _NG), lambda i: (0, 0)),
    pl.BlockSpec((_BP, 4), lambda i: (i, 0)),
    pl.BlockSpec((_BP, _C), lambda i: (i, 0)),
    pl.BlockSpec((_NG, 8), lambda i: (0, 0)),
]


@functools.partial(jax.jit, static_argnames=())
def kernel(pred_class_logits, pred_bounding_boxes, gt_class, gt_bounding_boxes):
    gt_t = jnp.zeros((8, _NG), jnp.float32).at[:4].set(gt_bounding_boxes[0].T)
    labc = jnp.broadcast_to(
        gt_class[0].astype(jnp.int32)[:, None], (_NG, 8))
    args = (gt_t, pred_bounding_boxes, pred_class_logits, labc)

    main, count = pl.pallas_call(
        _main_body,
        grid=(_NB,),
        in_specs=_IN_SPECS,
        out_specs=[pl.BlockSpec((1, 1), lambda i: (0, 0)),
                   pl.BlockSpec((1, 1), lambda i: (0, 0))],
        out_shape=[jax.ShapeDtypeStruct((1, 1), jnp.float32),
                   jax.ShapeDtypeStruct((1, 1), jnp.float32)],
        scratch_shapes=[pltpu.VMEM((1, 1), jnp.float32)] * 4
        + [pltpu.VMEM((_NG, _C), jnp.bfloat16)],
    )(*args)

    def _fallback(_):
        fb = pl.pallas_call(
            _fb_body,
            grid=(_NB,),
            in_specs=_IN_SPECS,
            out_specs=pl.BlockSpec((1, 1), lambda i: (0, 0)),
            out_shape=jax.ShapeDtypeStruct((1, 1), jnp.float32),
            scratch_shapes=[pltpu.VMEM((1, _NG), jnp.float32)] * 2,
        )(*args)
        return fb[0, 0]

    return jax.lax.cond(count[0, 0] > 0.0, lambda _: main[0, 0],
                        _fallback, None)
